# Initial kernel scaffold; baseline (speedup 1.0000x reference)
import jax
import jax.numpy as jnp
from jax.experimental import pallas as pl


def kernel(node_feats, edge_feats, edge_index, Wq, Wk, Wv, We):
    raise NotImplementedError("write your pallas kernel here")


# speedup vs baseline: 12.3117x; 12.3117x over previous
import functools

import jax
import jax.numpy as jnp
import numpy as np
from jax import lax
from jax.experimental import pallas as pl
from jax.experimental.pallas import tpu as pltpu
from jax.experimental.pallas import tpu_sc as plsc

NC = 2
NS = 16
LANES = 16

_GATHER_DNUMS = lax.GatherDimensionNumbers(
    offset_dims=(), collapsed_slice_dims=(0,), start_index_map=(0,))


def _lane_perm(v, idx):
    return lax.gather(v, idx[:, None], _GATHER_DNUMS, (1,),
                      mode=lax.GatherScatterMode.PROMISE_IN_BOUNDS)


def _lane_perm_i32(v, idx):
    return _lane_perm(v.view(jnp.float32), idx).view(jnp.int32)


def _proj3_body(x_ref, wk_ref, wq_ref, wv_ref, k_ref, q_ref, v_ref):
    x = x_ref[...]
    k_ref[...] = jnp.dot(x, wk_ref[...], preferred_element_type=jnp.float32)
    q_ref[...] = jnp.dot(x, wq_ref[...], preferred_element_type=jnp.float32)
    v_ref[...] = jnp.dot(x, wv_ref[...], preferred_element_type=jnp.float32)


def _proj1_body(x_ref, w_ref, o_ref):
    o_ref[...] = jnp.dot(x_ref[...], w_ref[...], preferred_element_type=jnp.float32)


def _node_proj(x, wkT, wqT, wvT, bm):
    r, f = x.shape
    hd = wkT.shape[1]
    grid = r // bm
    out = jax.ShapeDtypeStruct((r, hd), jnp.float32)
    wspec = pl.BlockSpec((f, hd), lambda i: (0, 0))
    return pl.pallas_call(
        _proj3_body,
        grid=(grid,),
        in_specs=[pl.BlockSpec((bm, f), lambda i: (i, 0)), wspec, wspec, wspec],
        out_specs=[pl.BlockSpec((bm, hd), lambda i: (i, 0))] * 3,
        out_shape=[out, out, out],
    )(x, wkT, wqT, wvT)


def _edge_proj(x, weT, bm):
    r, f = x.shape
    hd = weT.shape[1]
    grid = r // bm
    return pl.pallas_call(
        _proj1_body,
        grid=(grid,),
        in_specs=[pl.BlockSpec((bm, f), lambda i: (i, 0)),
                  pl.BlockSpec((f, hd), lambda i: (0, 0))],
        out_specs=pl.BlockSpec((bm, hd), lambda i: (i, 0)),
        out_shape=jax.ShapeDtypeStruct((r, hd), jnp.float32),
    )(x, weT)


def _make_edge_kernel(n, e_pad, hd, heads, d, b):
    e_per_w = e_pad // (NC * NS)
    nb = e_per_w // b
    chunk = NS * LANES * 8
    n_pad = ((n + chunk - 1) // chunk) * chunk
    rows_per_tile = n_pad // NS
    zrows = n_pad // LANES
    zrows_per_tile = zrows // NS
    assert e_per_w % b == 0 and b % LANES == 0
    assert b >= zrows_per_tile and zrows % NS == 0 and zrows_per_tile % 8 == 0

    mesh = plsc.VectorSubcoreMesh(core_axis_name="c", subcore_axis_name="s")

    @functools.partial(
        pl.kernel,
        out_type=(jax.ShapeDtypeStruct((e_pad, hd), jnp.float32),
                  jax.ShapeDtypeStruct((NC, n_pad, hd), jnp.float32),
                  jax.ShapeDtypeStruct((NC, zrows, hd), jnp.float32)),
        mesh=mesh,
        compiler_params=pltpu.CompilerParams(needs_layout_passes=False),
        scratch_types=[
            pltpu.VMEM((b,), jnp.int32),
            pltpu.VMEM((b,), jnp.int32),
            pltpu.VMEM((b,), jnp.int32),
            pltpu.VMEM((b, hd), jnp.float32),
            pltpu.VMEM((b, hd), jnp.float32),
            pltpu.VMEM((b, hd), jnp.float32),
            pltpu.VMEM((b, hd), jnp.float32),
            pltpu.VMEM((b, hd), jnp.float32),
            pltpu.VMEM_SHARED((n_pad, hd), jnp.float32),
            pltpu.VMEM_SHARED((zrows, hd), jnp.float32),
            pltpu.SemaphoreType.DMA,
            pltpu.SemaphoreType.DMA,
            pltpu.SemaphoreType.DMA,
            pltpu.SemaphoreType.DMA,
        ],
    )
    def edge_kernel(k_hbm, q_hbm, v_hbm, pe_hbm, src_hbm, dst_hbm,
                    eout_hbm, wv_hbm, z_hbm,
                    src_v, dst_v, idxz, kg, qg, vg, peb, zrb,
                    accum, zacc, sem1, sem2, sem3, sem4):
        msgb = kg
        eoutb = qg
        cid = lax.axis_index("c")
        sid = lax.axis_index("s")
        wid = sid * NC + cid
        lane = lax.iota(jnp.int32, LANES)
        zero16 = jnp.zeros((LANES,), jnp.float32)

        def pb_zero(r, _):
            for j in range(hd // LANES):
                peb[r, pl.ds(LANES * j, LANES)] = zero16
            return 0
        lax.fori_loop(0, b, pb_zero, 0)

        row0 = sid * rows_per_tile
        left = rows_per_tile
        while left > 0:
            step = min(b, left)
            pltpu.sync_copy(peb.at[pl.ds(0, step)],
                            accum.at[pl.ds(row0 + rows_per_tile - left, step)])
            left -= step
        pltpu.sync_copy(peb.at[pl.ds(0, zrows_per_tile)],
                        zacc.at[pl.ds(sid * zrows_per_tile, zrows_per_tile)])
        plsc.subcore_barrier()

        def block_body(blk, _):
            base = wid * e_per_w + blk * b
            pltpu.sync_copy(src_hbm.at[pl.ds(base, b)], src_v)
            pltpu.sync_copy(dst_hbm.at[pl.ds(base, b)], dst_v)
            cp1 = pltpu.async_copy(k_hbm.at[src_v], kg, sem1)
            cp2 = pltpu.async_copy(q_hbm.at[dst_v], qg, sem2)
            cp3 = pltpu.async_copy(v_hbm.at[src_v], vg, sem3)
            cp4 = pltpu.async_copy(pe_hbm.at[pl.ds(base, b)], peb, sem4)
            cp1.wait()
            cp2.wait()
            cp3.wait()
            cp4.wait()

            def group_body(g, _):
                g0 = g * LANES
                dd16 = dst_v[pl.ds(g0, LANES)]
                idxz[pl.ds(g0, LANES)] = jnp.right_shift(dd16, 4)

                def edge_body(j, _):
                    i = g0 + j
                    zrow = zero16
                    for h in range(heads):
                        sl = pl.ds(d * h, d)
                        sc = jnp.clip(kg[i, sl] * qg[i, sl] * (1.0 / np.sqrt(d)),
                                      -5.0, 5.0) * peb[i, sl]
                        tot = sc
                        for sh in (8, 4, 2, 1):
                            tot = tot + _lane_perm(tot, lane ^ sh)
                        eoutb[i, sl] = sc
                        s = jnp.exp(jnp.clip(tot, -5.0, 5.0))
                        msgb[i, sl] = vg[i, sl] * s
                        zrow = jnp.where(lane == h, s, zrow)
                    dsel = _lane_perm_i32(dd16, jnp.full((LANES,), j, jnp.int32))
                    c = dsel & 15
                    cc = jnp.right_shift(c, 1)
                    odd = (c & 1) == 1
                    zhi = jnp.where(lane >= 8, _lane_perm(zrow, lane & 7), 0.0)
                    sel = jnp.where(odd, zhi, zrow)
                    for jc in range(hd // LANES):
                        zrb[i, pl.ds(LANES * jc, LANES)] = jnp.where(
                            cc == jc, sel, 0.0)
                    return 0

                lax.fori_loop(0, LANES, edge_body, 0)
                return 0

            lax.fori_loop(0, b // LANES, group_body, 0)

            pltpu.sync_copy(eoutb, eout_hbm.at[pl.ds(base, b)])
            pltpu.sync_copy(msgb, accum.at[dst_v], add=True)
            pltpu.sync_copy(zrb, zacc.at[idxz], add=True)
            return 0

        lax.fori_loop(0, nb, block_body, 0)
        plsc.subcore_barrier()

        pltpu.sync_copy(accum.at[pl.ds(row0, rows_per_tile)],
                        wv_hbm.at[cid, pl.ds(row0, rows_per_tile)])
        zr0 = sid * zrows_per_tile
        pltpu.sync_copy(zacc.at[pl.ds(zr0, zrows_per_tile)],
                        z_hbm.at[cid, pl.ds(zr0, zrows_per_tile)])

    return edge_kernel


def _epi_body(a_ref, b_ref, za_ref, zb_ref, ex_ref, o_ref):
    wv = a_ref[...] + b_ref[...]
    z = za_ref[...] + zb_ref[...]
    zrep = jnp.dot(z, ex_ref[...], preferred_element_type=jnp.float32)
    o_ref[...] = wv / (zrep + 1e-6)


def _epilogue(wv2, z2, n_pad, heads, d, bm):
    hd = heads * d
    expand = jnp.asarray(np.repeat(np.eye(heads, dtype=np.float32), d, axis=1))
    grid = n_pad // bm
    return pl.pallas_call(
        _epi_body,
        grid=(grid,),
        in_specs=[pl.BlockSpec((bm, hd), lambda i: (i, 0)),
                  pl.BlockSpec((bm, hd), lambda i, nb=grid: (i + nb, 0)),
                  pl.BlockSpec((bm, heads), lambda i: (i, 0)),
                  pl.BlockSpec((bm, heads), lambda i, nb=grid: (i + nb, 0)),
                  pl.BlockSpec((heads, hd), lambda i: (0, 0))],
        out_specs=pl.BlockSpec((bm, hd), lambda i: (i, 0)),
        out_shape=jax.ShapeDtypeStruct((n_pad, hd), jnp.float32),
    )(wv2, wv2, z2, z2, expand)


def kernel(node_feats, edge_feats, edge_index, Wq, Wk, Wv, We):
    n, f = node_feats.shape
    e = edge_feats.shape[0]
    hd = Wq.shape[0]
    heads, d = 8, hd // 8

    k_t, q_t, v_t = _node_proj(node_feats, Wk.T, Wq.T, Wv.T, bm=2000)
    pe = _edge_proj(edge_feats, We.T, bm=2000)

    b = 48
    nw = NC * NS
    e_per_w = -(-e // (nw * b)) * b
    e_pad = e_per_w * nw
    chunkn = NS * LANES * 8
    n_pad = ((n + chunkn - 1) // chunkn) * chunkn
    pad = e_pad - e
    src = jnp.concatenate([edge_index[0], jnp.zeros((pad,), edge_index.dtype)])
    dst = jnp.concatenate(
        [edge_index[1], jnp.full((pad,), n_pad - 1, edge_index.dtype)])
    pe = jnp.concatenate([pe, jnp.zeros((pad, hd), jnp.float32)])

    edge_kernel = _make_edge_kernel(n, e_pad, hd, heads, d, b=b)
    e_out, wv_part, z_part = edge_kernel(k_t, q_t, v_t, pe, src, dst)

    wv2 = wv_part.reshape(NC * n_pad, hd)
    z2 = z_part.reshape(NC * n_pad, heads)
    h_out = _epilogue(wv2, z2, n_pad, heads, d, bm=2048)

    return h_out[:n].reshape(n, heads, d), e_out[:e].reshape(e, heads, d)


# speedup vs baseline: 13.5966x; 1.1044x over previous
import functools

import jax
import jax.numpy as jnp
import numpy as np
from jax import lax
from jax.experimental import pallas as pl
from jax.experimental.pallas import tpu as pltpu
from jax.experimental.pallas import tpu_sc as plsc

NC = 2
NS = 16
LANES = 16

_GATHER_DNUMS = lax.GatherDimensionNumbers(
    offset_dims=(), collapsed_slice_dims=(0,), start_index_map=(0,))


def _lane_perm(v, idx):
    return lax.gather(v, idx[:, None], _GATHER_DNUMS, (1,),
                      mode=lax.GatherScatterMode.PROMISE_IN_BOUNDS)


def _lane_perm_i32(v, idx):
    return _lane_perm(v.view(jnp.float32), idx).view(jnp.int32)


def _proj3_body(x_ref, wk_ref, wq_ref, wv_ref, k_ref, q_ref, v_ref):
    x = x_ref[...]
    k_ref[...] = jnp.dot(x, wk_ref[...], preferred_element_type=jnp.float32)
    q_ref[...] = jnp.dot(x, wq_ref[...], preferred_element_type=jnp.float32)
    v_ref[...] = jnp.dot(x, wv_ref[...], preferred_element_type=jnp.float32)


def _proj1_body(x_ref, w_ref, o_ref):
    o_ref[...] = jnp.dot(x_ref[...], w_ref[...], preferred_element_type=jnp.float32)


def _node_proj(x, wkT, wqT, wvT, bm):
    r, f = x.shape
    hd = wkT.shape[1]
    grid = r // bm
    out = jax.ShapeDtypeStruct((r, hd), jnp.float32)
    wspec = pl.BlockSpec((f, hd), lambda i: (0, 0))
    return pl.pallas_call(
        _proj3_body,
        grid=(grid,),
        in_specs=[pl.BlockSpec((bm, f), lambda i: (i, 0)), wspec, wspec, wspec],
        out_specs=[pl.BlockSpec((bm, hd), lambda i: (i, 0))] * 3,
        out_shape=[out, out, out],
    )(x, wkT, wqT, wvT)


def _edge_proj(x, weT, bm):
    r, f = x.shape
    hd = weT.shape[1]
    grid = r // bm
    return pl.pallas_call(
        _proj1_body,
        grid=(grid,),
        in_specs=[pl.BlockSpec((bm, f), lambda i: (i, 0)),
                  pl.BlockSpec((f, hd), lambda i: (0, 0))],
        out_specs=pl.BlockSpec((bm, hd), lambda i: (i, 0)),
        out_shape=jax.ShapeDtypeStruct((r, hd), jnp.float32),
    )(x, weT)


def _make_edge_kernel(n, e_pad, hd, heads, d, b):
    e_per_w = e_pad // (NC * NS)
    nb = e_per_w // b
    chunk = NS * LANES * 8
    n_pad = ((n + chunk - 1) // chunk) * chunk
    rows_per_tile = n_pad // NS
    zrows = n_pad // LANES
    zrows_per_tile = zrows // NS
    assert e_per_w % b == 0 and b % LANES == 0
    assert zrows % NS == 0 and zrows_per_tile % 8 == 0

    assert nb % 2 == 0
    npairs = nb // 2

    mesh = plsc.VectorSubcoreMesh(core_axis_name="c", subcore_axis_name="s")

    slot_scratch = [
        pltpu.VMEM((b,), jnp.int32),
        pltpu.VMEM((b,), jnp.int32),
        pltpu.VMEM((b,), jnp.int32),
        pltpu.VMEM((b, hd), jnp.float32),
        pltpu.VMEM((b, hd), jnp.float32),
        pltpu.VMEM((b, hd), jnp.float32),
        pltpu.VMEM((b, hd), jnp.float32),
        pltpu.VMEM((b, hd), jnp.float32),
        pltpu.SemaphoreType.DMA,
        pltpu.SemaphoreType.DMA,
        pltpu.SemaphoreType.DMA,
        pltpu.SemaphoreType.DMA,
        pltpu.SemaphoreType.DMA,
        pltpu.SemaphoreType.DMA,
        pltpu.SemaphoreType.DMA,
    ]

    @functools.partial(
        pl.kernel,
        out_type=(jax.ShapeDtypeStruct((e_pad, hd), jnp.float32),
                  jax.ShapeDtypeStruct((NC, n_pad, hd), jnp.float32),
                  jax.ShapeDtypeStruct((NC, zrows, hd), jnp.float32)),
        mesh=mesh,
        compiler_params=pltpu.CompilerParams(needs_layout_passes=False),
        scratch_types=slot_scratch + slot_scratch + [
            pltpu.VMEM_SHARED((n_pad, hd), jnp.float32),
            pltpu.VMEM_SHARED((zrows, hd), jnp.float32),
        ],
    )
    def edge_kernel(k_hbm, q_hbm, v_hbm, pe_hbm, src_hbm, dst_hbm,
                    eout_hbm, wv_hbm, z_hbm, *scr):
        ns = 15
        slots = (scr[:ns], scr[ns:2 * ns])
        accum, zacc = scr[2 * ns], scr[2 * ns + 1]
        cid = lax.axis_index("c")
        sid = lax.axis_index("s")
        wid = sid * NC + cid
        lane = lax.iota(jnp.int32, LANES)
        zero16 = jnp.zeros((LANES,), jnp.float32)

        def issue_gathers(t, sl):
            src_v, dst_v, _, kg, qg, vg, peb, _, sk, sq, sv, sp, _, _, _ = sl
            base = wid * e_per_w + t * b
            pltpu.sync_copy(src_hbm.at[pl.ds(base, b)], src_v)
            pltpu.sync_copy(dst_hbm.at[pl.ds(base, b)], dst_v)
            pltpu.async_copy(k_hbm.at[src_v], kg, sk)
            pltpu.async_copy(q_hbm.at[dst_v], qg, sq)
            pltpu.async_copy(v_hbm.at[src_v], vg, sv)
            pltpu.async_copy(pe_hbm.at[pl.ds(base, b)], peb, sp)

        def wait_gathers(sl):
            src_v, dst_v, _, kg, qg, vg, peb, _, sk, sq, sv, sp, _, _, _ = sl
            pltpu.make_async_copy(k_hbm.at[src_v], kg, sk).wait()
            pltpu.make_async_copy(q_hbm.at[dst_v], qg, sq).wait()
            pltpu.make_async_copy(v_hbm.at[src_v], vg, sv).wait()
            pltpu.make_async_copy(pe_hbm.at[pl.ds(0, b)], peb, sp).wait()

        def issue_outputs(t, sl):
            _, dst_v, idxz, kg, qg, _, _, zrb, _, _, _, _, se, sm, sz = sl
            base = wid * e_per_w + t * b
            pltpu.async_copy(qg, eout_hbm.at[pl.ds(base, b)], se)
            pltpu.async_copy(kg, accum.at[dst_v], sm, add=True)
            pltpu.async_copy(zrb, zacc.at[idxz], sz, add=True)

        def wait_outputs(sl):
            _, dst_v, idxz, kg, qg, _, _, zrb, _, _, _, _, se, sm, sz = sl
            pltpu.make_async_copy(qg, eout_hbm.at[pl.ds(0, b)], se).wait()
            pltpu.make_async_copy(kg, accum.at[dst_v], sm).wait()
            pltpu.make_async_copy(zrb, zacc.at[idxz], sz).wait()

        def compute(sl):
            _, dst_v, idxz, kg, qg, vg, peb, zrb = sl[:8]
            msgb = kg
            eoutb = qg

            def group_body(g, _):
                g0 = g * LANES
                dd16 = dst_v[pl.ds(g0, LANES)]
                idxz[pl.ds(g0, LANES)] = jnp.right_shift(dd16, 4)

                def edge_body(j, _):
                    i = g0 + j
                    zrow = zero16
                    for h in range(heads):
                        sl_ = pl.ds(d * h, d)
                        sc = jnp.clip(kg[i, sl_] * qg[i, sl_] * (1.0 / np.sqrt(d)),
                                      -5.0, 5.0) * peb[i, sl_]
                        tot = sc
                        for sh in (8, 4, 2, 1):
                            tot = tot + _lane_perm(tot, lane ^ sh)
                        eoutb[i, sl_] = sc
                        s = jnp.exp(jnp.clip(tot, -5.0, 5.0))
                        msgb[i, sl_] = vg[i, sl_] * s
                        zrow = jnp.where(lane == h, s, zrow)
                    dsel = _lane_perm_i32(dd16, jnp.full((LANES,), j, jnp.int32))
                    c = dsel & 15
                    cc = jnp.right_shift(c, 1)
                    odd = (c & 1) == 1
                    zhi = jnp.where(lane >= 8, _lane_perm(zrow, lane & 7), 0.0)
                    sel = jnp.where(odd, zhi, zrow)
                    for jc in range(hd // LANES):
                        zrb[i, pl.ds(LANES * jc, LANES)] = jnp.where(
                            cc == jc, sel, 0.0)
                    return 0

                lax.fori_loop(0, LANES, edge_body, 0)
                return 0

            lax.fori_loop(0, b // LANES, group_body, 0)

        peb0 = slots[0][6]

        def pb_zero(r, _):
            for j in range(hd // LANES):
                peb0[r, pl.ds(LANES * j, LANES)] = zero16
            return 0
        lax.fori_loop(0, b, pb_zero, 0)

        row0 = sid * rows_per_tile
        left = rows_per_tile
        while left > 0:
            step = min(b, left)
            pltpu.sync_copy(peb0.at[pl.ds(0, step)],
                            accum.at[pl.ds(row0 + rows_per_tile - left, step)])
            left -= step
        left = zrows_per_tile
        zoff = sid * zrows_per_tile
        while left > 0:
            step = min(b, left)
            pltpu.sync_copy(peb0.at[pl.ds(0, step)],
                            zacc.at[pl.ds(zoff + zrows_per_tile - left, step)])
            left -= step
        plsc.subcore_barrier()

        s0, s1 = slots
        issue_gathers(0, s0)

        wait_gathers(s0)
        issue_gathers(1, s1)
        compute(s0)
        issue_outputs(0, s0)
        wait_gathers(s1)
        wait_outputs(s0)
        issue_gathers(2, s0)
        compute(s1)
        issue_outputs(1, s1)

        def pair_body(p, _):
            t0 = 2 * p
            wait_outputs(s1)
            wait_gathers(s0)
            issue_gathers(t0 + 1, s1)
            compute(s0)
            issue_outputs(t0, s0)
            wait_gathers(s1)
            wait_outputs(s0)
            issue_gathers(t0 + 2, s0)
            compute(s1)
            issue_outputs(t0 + 1, s1)
            return 0

        lax.fori_loop(1, npairs - 1, pair_body, 0)

        wait_outputs(s1)
        wait_gathers(s0)
        issue_gathers(nb - 1, s1)
        compute(s0)
        issue_outputs(nb - 2, s0)
        wait_gathers(s1)
        wait_outputs(s0)
        compute(s1)
        issue_outputs(nb - 1, s1)
        wait_outputs(s1)
        plsc.subcore_barrier()

        pltpu.sync_copy(accum.at[pl.ds(row0, rows_per_tile)],
                        wv_hbm.at[cid, pl.ds(row0, rows_per_tile)])
        zr0 = sid * zrows_per_tile
        pltpu.sync_copy(zacc.at[pl.ds(zr0, zrows_per_tile)],
                        z_hbm.at[cid, pl.ds(zr0, zrows_per_tile)])

    return edge_kernel


def _epi_body(a_ref, b_ref, za_ref, zb_ref, ex_ref, o_ref):
    wv = a_ref[...] + b_ref[...]
    z = za_ref[...] + zb_ref[...]
    zrep = jnp.dot(z, ex_ref[...], preferred_element_type=jnp.float32)
    o_ref[...] = wv / (zrep + 1e-6)


def _epilogue(wv2, z2, n_pad, heads, d, bm):
    hd = heads * d
    expand = jnp.asarray(np.repeat(np.eye(heads, dtype=np.float32), d, axis=1))
    grid = n_pad // bm
    return pl.pallas_call(
        _epi_body,
        grid=(grid,),
        in_specs=[pl.BlockSpec((bm, hd), lambda i: (i, 0)),
                  pl.BlockSpec((bm, hd), lambda i, nb=grid: (i + nb, 0)),
                  pl.BlockSpec((bm, heads), lambda i: (i, 0)),
                  pl.BlockSpec((bm, heads), lambda i, nb=grid: (i + nb, 0)),
                  pl.BlockSpec((heads, hd), lambda i: (0, 0))],
        out_specs=pl.BlockSpec((bm, hd), lambda i: (i, 0)),
        out_shape=jax.ShapeDtypeStruct((n_pad, hd), jnp.float32),
    )(wv2, wv2, z2, z2, expand)


def kernel(node_feats, edge_feats, edge_index, Wq, Wk, Wv, We):
    n, f = node_feats.shape
    e = edge_feats.shape[0]
    hd = Wq.shape[0]
    heads, d = 8, hd // 8

    k_t, q_t, v_t = _node_proj(node_feats, Wk.T, Wq.T, Wv.T, bm=2000)
    pe = _edge_proj(edge_feats, We.T, bm=2000)

    b = 32
    nw = NC * NS
    e_per_w = -(-e // (nw * 2 * b)) * 2 * b
    e_pad = e_per_w * nw
    chunkn = NS * LANES * 8
    n_pad = ((n + chunkn - 1) // chunkn) * chunkn
    pad = e_pad - e
    src = jnp.concatenate([edge_index[0], jnp.zeros((pad,), edge_index.dtype)])
    dst = jnp.concatenate(
        [edge_index[1], jnp.full((pad,), n_pad - 1, edge_index.dtype)])
    pe = jnp.concatenate([pe, jnp.zeros((pad, hd), jnp.float32)])

    edge_kernel = _make_edge_kernel(n, e_pad, hd, heads, d, b=b)
    e_out, wv_part, z_part = edge_kernel(k_t, q_t, v_t, pe, src, dst)

    wv2 = wv_part.reshape(NC * n_pad, hd)
    z2 = z_part.reshape(NC * n_pad, heads)
    h_out = _epilogue(wv2, z2, n_pad, heads, d, bm=2048)

    return h_out[:n].reshape(n, heads, d), e_out[:e].reshape(e, heads, d)


# speedup vs baseline: 14.7731x; 1.0865x over previous
import functools

import jax
import jax.numpy as jnp
import numpy as np
from jax import lax
from jax.experimental import pallas as pl
from jax.experimental.pallas import tpu as pltpu
from jax.experimental.pallas import tpu_sc as plsc

NC = 2
NS = 16
LANES = 16

_GATHER_DNUMS = lax.GatherDimensionNumbers(
    offset_dims=(), collapsed_slice_dims=(0,), start_index_map=(0,))


def _lane_perm(v, idx):
    return lax.gather(v, idx[:, None], _GATHER_DNUMS, (1,),
                      mode=lax.GatherScatterMode.PROMISE_IN_BOUNDS)


def _lane_perm_i32(v, idx):
    return _lane_perm(v.view(jnp.float32), idx).view(jnp.int32)


def _proj3_body(x_ref, wk_ref, wq_ref, wv_ref, k_ref, q_ref, v_ref):
    x = x_ref[...]
    k_ref[...] = jnp.dot(x, wk_ref[...], preferred_element_type=jnp.float32)
    q_ref[...] = jnp.dot(x, wq_ref[...], preferred_element_type=jnp.float32)
    v_ref[...] = jnp.dot(x, wv_ref[...], preferred_element_type=jnp.float32)


def _proj1_body(x_ref, w_ref, o_ref):
    o_ref[...] = jnp.dot(x_ref[...], w_ref[...], preferred_element_type=jnp.float32)


def _node_proj(x, wkT, wqT, wvT, bm):
    r, f = x.shape
    hd = wkT.shape[1]
    grid = r // bm
    out = jax.ShapeDtypeStruct((r, hd), jnp.float32)
    wspec = pl.BlockSpec((f, hd), lambda i: (0, 0))
    return pl.pallas_call(
        _proj3_body,
        grid=(grid,),
        in_specs=[pl.BlockSpec((bm, f), lambda i: (i, 0)), wspec, wspec, wspec],
        out_specs=[pl.BlockSpec((bm, hd), lambda i: (i, 0))] * 3,
        out_shape=[out, out, out],
    )(x, wkT, wqT, wvT)


def _edge_proj(x, weT, bm):
    r, f = x.shape
    hd = weT.shape[1]
    grid = r // bm
    return pl.pallas_call(
        _proj1_body,
        grid=(grid,),
        in_specs=[pl.BlockSpec((bm, f), lambda i: (i, 0)),
                  pl.BlockSpec((f, hd), lambda i: (0, 0))],
        out_specs=pl.BlockSpec((bm, hd), lambda i: (i, 0)),
        out_shape=jax.ShapeDtypeStruct((r, hd), jnp.float32),
    )(x, weT)


def _make_edge_kernel(n, e_pad, hd, heads, d, b):
    e_per_w = e_pad // (NC * NS)
    nb = e_per_w // b
    chunk = NS * LANES * 8
    n_pad = ((n + chunk - 1) // chunk) * chunk
    rows_per_tile = n_pad // NS
    zrows = n_pad // LANES
    zrows_per_tile = zrows // NS
    assert e_per_w % b == 0 and b % LANES == 0
    assert zrows % NS == 0 and zrows_per_tile % 8 == 0

    assert nb % 2 == 0
    npairs = nb // 2

    mesh = plsc.VectorSubcoreMesh(core_axis_name="c", subcore_axis_name="s")

    slot_scratch = [
        pltpu.VMEM((b,), jnp.int32),
        pltpu.VMEM((b,), jnp.int32),
        pltpu.VMEM((b,), jnp.int32),
        pltpu.VMEM((b,), jnp.int32),
        pltpu.VMEM((b, hd), jnp.float32),
        pltpu.VMEM((b, hd), jnp.float32),
        pltpu.VMEM((b, hd), jnp.float32),
        pltpu.VMEM((b, hd), jnp.float32),
        pltpu.VMEM((b, hd), jnp.float32),
        pltpu.SemaphoreType.DMA,
        pltpu.SemaphoreType.DMA,
        pltpu.SemaphoreType.DMA,
        pltpu.SemaphoreType.DMA,
        pltpu.SemaphoreType.DMA,
        pltpu.SemaphoreType.DMA,
        pltpu.SemaphoreType.DMA,
        pltpu.SemaphoreType.DMA,
    ]

    @functools.partial(
        pl.kernel,
        out_type=(jax.ShapeDtypeStruct((e_pad, hd), jnp.float32),
                  jax.ShapeDtypeStruct((NC, n_pad, hd), jnp.float32),
                  jax.ShapeDtypeStruct((NC, zrows, hd), jnp.float32)),
        mesh=mesh,
        compiler_params=pltpu.CompilerParams(needs_layout_passes=False),
        scratch_types=slot_scratch + slot_scratch + [
            pltpu.VMEM_SHARED((n_pad, hd), jnp.float32),
            pltpu.VMEM_SHARED((zrows, hd), jnp.float32),
        ],
    )
    def edge_kernel(k_hbm, q_hbm, v_hbm, pe_hbm, src_hbm, dst_hbm,
                    eout_hbm, wv_hbm, z_hbm, *scr):
        ns = 17
        slots = (scr[:ns], scr[ns:2 * ns])
        accum, zacc = scr[2 * ns], scr[2 * ns + 1]
        cid = lax.axis_index("c")
        sid = lax.axis_index("s")
        wid = sid * NC + cid
        lane = lax.iota(jnp.int32, LANES)
        zero16 = jnp.zeros((LANES,), jnp.float32)

        def issue_idx(t, sl):
            idxs, idxd = sl[0], sl[1]
            si = sl[9]
            base = wid * e_per_w + t * b
            pltpu.async_copy(src_hbm.at[pl.ds(base, b)], idxs, si)
            pltpu.async_copy(dst_hbm.at[pl.ds(base, b)], idxd, si)

        def issue_gathers(t, sl):
            idxs, idxd, _, _, kg, qg, vg, peb = sl[:8]
            si, sk, sq, sv, sp = sl[9:14]
            base = wid * e_per_w + t * b
            pltpu.make_async_copy(src_hbm.at[pl.ds(0, b)], idxs, si).wait()
            pltpu.make_async_copy(dst_hbm.at[pl.ds(0, b)], idxd, si).wait()
            pltpu.async_copy(k_hbm.at[idxs], kg, sk)
            pltpu.async_copy(q_hbm.at[idxd], qg, sq)
            pltpu.async_copy(v_hbm.at[idxs], vg, sv)
            pltpu.async_copy(pe_hbm.at[pl.ds(base, b)], peb, sp)

        def wait_gathers(sl):
            idxs, idxd, _, _, kg, qg, vg, peb = sl[:8]
            sk, sq, sv, sp = sl[10:14]
            pltpu.make_async_copy(k_hbm.at[idxs], kg, sk).wait()
            pltpu.make_async_copy(q_hbm.at[idxd], qg, sq).wait()
            pltpu.make_async_copy(v_hbm.at[idxs], vg, sv).wait()
            pltpu.make_async_copy(pe_hbm.at[pl.ds(0, b)], peb, sp).wait()

        def issue_outputs(t, sl):
            _, _, dst_v, idxz, kg, qg, _, _, zrb = sl[:9]
            se, sm, sz = sl[14:17]
            base = wid * e_per_w + t * b
            pltpu.async_copy(qg, eout_hbm.at[pl.ds(base, b)], se)
            pltpu.async_copy(kg, accum.at[dst_v], sm, add=True)
            pltpu.async_copy(zrb, zacc.at[idxz], sz, add=True)

        def wait_outputs(sl):
            _, _, dst_v, idxz, kg, qg, _, _, zrb = sl[:9]
            se, sm, sz = sl[14:17]
            pltpu.make_async_copy(qg, eout_hbm.at[pl.ds(0, b)], se).wait()
            pltpu.make_async_copy(kg, accum.at[dst_v], sm).wait()
            pltpu.make_async_copy(zrb, zacc.at[idxz], sz).wait()

        def stage_idx(sl):
            _, idxd, dst_v, idxz = sl[:4]
            for g in range(b // LANES):
                g0 = g * LANES
                dd16 = idxd[pl.ds(g0, LANES)]
                dst_v[pl.ds(g0, LANES)] = dd16
                idxz[pl.ds(g0, LANES)] = jnp.right_shift(dd16, 4)

        def compute(sl):
            _, _, dst_v, idxz, kg, qg, vg, peb, zrb = sl[:9]
            msgb = kg
            eoutb = qg

            def group_body(g, _):
                g0 = g * LANES
                dd16 = dst_v[pl.ds(g0, LANES)]

                def edge_body(j, _):
                    i = g0 + j
                    zrow = zero16
                    for h in range(heads):
                        sl_ = pl.ds(d * h, d)
                        sc = jnp.clip(kg[i, sl_] * qg[i, sl_] * (1.0 / np.sqrt(d)),
                                      -5.0, 5.0) * peb[i, sl_]
                        tot = sc
                        for sh in (8, 4, 2, 1):
                            tot = tot + _lane_perm(tot, lane ^ sh)
                        eoutb[i, sl_] = sc
                        s = jnp.exp(jnp.clip(tot, -5.0, 5.0))
                        msgb[i, sl_] = vg[i, sl_] * s
                        zrow = jnp.where(lane == h, s, zrow)
                    dsel = _lane_perm_i32(dd16, jnp.full((LANES,), j, jnp.int32))
                    c = dsel & 15
                    cc = jnp.right_shift(c, 1)
                    odd = (c & 1) == 1
                    zhi = jnp.where(lane >= 8, _lane_perm(zrow, lane & 7), 0.0)
                    sel = jnp.where(odd, zhi, zrow)
                    for jc in range(hd // LANES):
                        zrb[i, pl.ds(LANES * jc, LANES)] = jnp.where(
                            cc == jc, sel, 0.0)
                    return 0

                lax.fori_loop(0, LANES, edge_body, 0)
                return 0

            lax.fori_loop(0, b // LANES, group_body, 0)

        peb0 = slots[0][7]

        def pb_zero(r, _):
            for j in range(hd // LANES):
                peb0[r, pl.ds(LANES * j, LANES)] = zero16
            return 0
        lax.fori_loop(0, b, pb_zero, 0)

        row0 = sid * rows_per_tile
        left = rows_per_tile
        while left > 0:
            step = min(b, left)
            pltpu.sync_copy(peb0.at[pl.ds(0, step)],
                            accum.at[pl.ds(row0 + rows_per_tile - left, step)])
            left -= step
        left = zrows_per_tile
        zoff = sid * zrows_per_tile
        while left > 0:
            step = min(b, left)
            pltpu.sync_copy(peb0.at[pl.ds(0, step)],
                            zacc.at[pl.ds(zoff + zrows_per_tile - left, step)])
            left -= step
        plsc.subcore_barrier()

        s0, s1 = slots
        issue_idx(0, s0)
        issue_idx(1, s1)
        issue_gathers(0, s0)

        wait_gathers(s0)
        stage_idx(s0)
        issue_idx(2, s0)
        issue_gathers(1, s1)
        compute(s0)
        issue_outputs(0, s0)
        wait_gathers(s1)
        stage_idx(s1)
        issue_idx(3, s1)
        wait_outputs(s0)
        issue_gathers(2, s0)
        compute(s1)
        issue_outputs(1, s1)

        def pair_body(p, _):
            t0 = 2 * p
            wait_outputs(s1)
            wait_gathers(s0)
            stage_idx(s0)
            issue_idx(t0 + 2, s0)
            issue_gathers(t0 + 1, s1)
            compute(s0)
            issue_outputs(t0, s0)
            wait_gathers(s1)
            stage_idx(s1)
            issue_idx(t0 + 3, s1)
            wait_outputs(s0)
            issue_gathers(t0 + 2, s0)
            compute(s1)
            issue_outputs(t0 + 1, s1)
            return 0

        lax.fori_loop(1, npairs - 1, pair_body, 0)

        wait_outputs(s1)
        wait_gathers(s0)
        stage_idx(s0)
        issue_gathers(nb - 1, s1)
        compute(s0)
        issue_outputs(nb - 2, s0)
        wait_gathers(s1)
        stage_idx(s1)
        wait_outputs(s0)
        compute(s1)
        issue_outputs(nb - 1, s1)
        wait_outputs(s1)
        plsc.subcore_barrier()

        pltpu.sync_copy(accum.at[pl.ds(row0, rows_per_tile)],
                        wv_hbm.at[cid, pl.ds(row0, rows_per_tile)])
        zr0 = sid * zrows_per_tile
        pltpu.sync_copy(zacc.at[pl.ds(zr0, zrows_per_tile)],
                        z_hbm.at[cid, pl.ds(zr0, zrows_per_tile)])

    return edge_kernel


def _epi_body(a_ref, b_ref, za_ref, zb_ref, ex_ref, o_ref):
    wv = a_ref[...] + b_ref[...]
    z = za_ref[...] + zb_ref[...]
    zrep = jnp.dot(z, ex_ref[...], preferred_element_type=jnp.float32)
    o_ref[...] = wv / (zrep + 1e-6)


def _epilogue(wv2, z2, n_pad, heads, d, bm):
    hd = heads * d
    expand = jnp.asarray(np.repeat(np.eye(heads, dtype=np.float32), d, axis=1))
    grid = n_pad // bm
    return pl.pallas_call(
        _epi_body,
        grid=(grid,),
        in_specs=[pl.BlockSpec((bm, hd), lambda i: (i, 0)),
                  pl.BlockSpec((bm, hd), lambda i, nb=grid: (i + nb, 0)),
                  pl.BlockSpec((bm, heads), lambda i: (i, 0)),
                  pl.BlockSpec((bm, heads), lambda i, nb=grid: (i + nb, 0)),
                  pl.BlockSpec((heads, hd), lambda i: (0, 0))],
        out_specs=pl.BlockSpec((bm, hd), lambda i: (i, 0)),
        out_shape=jax.ShapeDtypeStruct((n_pad, hd), jnp.float32),
    )(wv2, wv2, z2, z2, expand)


def kernel(node_feats, edge_feats, edge_index, Wq, Wk, Wv, We):
    n, f = node_feats.shape
    e = edge_feats.shape[0]
    hd = Wq.shape[0]
    heads, d = 8, hd // 8

    k_t, q_t, v_t = _node_proj(node_feats, Wk.T, Wq.T, Wv.T, bm=2000)
    pe = _edge_proj(edge_feats, We.T, bm=2000)

    b = 32
    nw = NC * NS
    e_per_w = -(-e // (nw * 2 * b)) * 2 * b
    e_pad = e_per_w * nw
    chunkn = NS * LANES * 8
    n_pad = ((n + chunkn - 1) // chunkn) * chunkn
    pad = e_pad - e
    src = jnp.concatenate([edge_index[0], jnp.zeros((pad,), edge_index.dtype)])
    dst = jnp.concatenate(
        [edge_index[1], jnp.full((pad,), n_pad - 1, edge_index.dtype)])
    pe = jnp.concatenate([pe, jnp.zeros((pad, hd), jnp.float32)])

    edge_kernel = _make_edge_kernel(n, e_pad, hd, heads, d, b=b)
    e_out, wv_part, z_part = edge_kernel(k_t, q_t, v_t, pe, src, dst)

    wv2 = wv_part.reshape(NC * n_pad, hd)
    z2 = z_part.reshape(NC * n_pad, heads)
    h_out = _epilogue(wv2, z2, n_pad, heads, d, bm=2048)

    return h_out[:n].reshape(n, heads, d), e_out[:e].reshape(e, heads, d)


# speedup vs baseline: 35.2974x; 2.3893x over previous
import functools

import jax
import jax.numpy as jnp
import numpy as np
from jax import lax
from jax.experimental import pallas as pl
from jax.experimental.pallas import tpu as pltpu
from jax.experimental.pallas import tpu_sc as plsc

NC = 2
NS = 16
LANES = 16

_GATHER_DNUMS = lax.GatherDimensionNumbers(
    offset_dims=(), collapsed_slice_dims=(0,), start_index_map=(0,))


def _lane_perm(v, idx):
    return lax.gather(v, idx[:, None], _GATHER_DNUMS, (1,),
                      mode=lax.GatherScatterMode.PROMISE_IN_BOUNDS)


def _lane_perm_i32(v, idx):
    return _lane_perm(v.view(jnp.float32), idx).view(jnp.int32)


def _proj3_body(x_ref, wk_ref, wq_ref, wv_ref, k_ref, q_ref, v_ref):
    x = x_ref[...]
    k_ref[...] = jnp.dot(x, wk_ref[...], preferred_element_type=jnp.float32)
    q_ref[...] = jnp.dot(x, wq_ref[...], preferred_element_type=jnp.float32)
    v_ref[...] = jnp.dot(x, wv_ref[...], preferred_element_type=jnp.float32)


def _proj1_body(x_ref, w_ref, o_ref):
    o_ref[...] = jnp.dot(x_ref[...], w_ref[...], preferred_element_type=jnp.float32)


def _node_proj(x, wkT, wqT, wvT, bm):
    r, f = x.shape
    hd = wkT.shape[1]
    grid = r // bm
    out = jax.ShapeDtypeStruct((r, hd), jnp.float32)
    wspec = pl.BlockSpec((f, hd), lambda i: (0, 0))
    return pl.pallas_call(
        _proj3_body,
        grid=(grid,),
        in_specs=[pl.BlockSpec((bm, f), lambda i: (i, 0)), wspec, wspec, wspec],
        out_specs=[pl.BlockSpec((bm, hd), lambda i: (i, 0))] * 3,
        out_shape=[out, out, out],
    )(x, wkT, wqT, wvT)


def _edge_proj(x, weT, bm):
    r, f = x.shape
    hd = weT.shape[1]
    grid = r // bm
    return pl.pallas_call(
        _proj1_body,
        grid=(grid,),
        in_specs=[pl.BlockSpec((bm, f), lambda i: (i, 0)),
                  pl.BlockSpec((f, hd), lambda i: (0, 0))],
        out_specs=pl.BlockSpec((bm, hd), lambda i: (i, 0)),
        out_shape=jax.ShapeDtypeStruct((r, hd), jnp.float32),
    )(x, weT)


def _make_edge_kernel(n, e_pad, hd, heads, d, b):
    e_per_w = e_pad // (NC * NS)
    nb = e_per_w // b
    chunk = NS * LANES * 8
    n_pad = ((n + chunk - 1) // chunk) * chunk
    rows_per_tile = n_pad // NS
    zrows = n_pad // LANES
    zrows_per_tile = zrows // NS
    assert e_per_w % b == 0 and b % LANES == 0
    assert zrows % NS == 0 and zrows_per_tile % 8 == 0

    assert nb % 2 == 0
    npairs = nb // 2

    mesh = plsc.VectorSubcoreMesh(core_axis_name="c", subcore_axis_name="s")

    slot_scratch = [
        pltpu.VMEM((b,), jnp.int32),
        pltpu.VMEM((b,), jnp.int32),
        pltpu.VMEM((b,), jnp.int32),
        pltpu.VMEM((b,), jnp.int32),
        pltpu.VMEM((b, hd), jnp.float32),
        pltpu.VMEM((b, hd), jnp.float32),
        pltpu.VMEM((b, hd), jnp.float32),
        pltpu.VMEM((b, hd), jnp.float32),
        pltpu.VMEM((b, hd), jnp.float32),
        pltpu.VMEM((b, hd), jnp.float32),
        pltpu.VMEM((b, hd), jnp.float32),
        pltpu.SemaphoreType.DMA,
        pltpu.SemaphoreType.DMA,
        pltpu.SemaphoreType.DMA,
        pltpu.SemaphoreType.DMA,
        pltpu.SemaphoreType.DMA,
        pltpu.SemaphoreType.DMA,
        pltpu.SemaphoreType.DMA,
        pltpu.SemaphoreType.DMA,
    ]

    @functools.partial(
        pl.kernel,
        out_type=(jax.ShapeDtypeStruct((e_pad, hd), jnp.float32),
                  jax.ShapeDtypeStruct((NC, n_pad, hd), jnp.float32),
                  jax.ShapeDtypeStruct((NC, zrows, hd), jnp.float32)),
        mesh=mesh,
        compiler_params=pltpu.CompilerParams(needs_layout_passes=False),
        scratch_types=slot_scratch + slot_scratch + [
            pltpu.VMEM_SHARED((n_pad, hd), jnp.float32),
            pltpu.VMEM_SHARED((zrows, hd), jnp.float32),
        ],
    )
    def edge_kernel(k_hbm, q_hbm, v_hbm, pe_hbm, src_hbm, dst_hbm,
                    eout_hbm, wv_hbm, z_hbm, *scr):
        ns = 19
        slots = (scr[:ns], scr[ns:2 * ns])
        accum, zacc = scr[2 * ns], scr[2 * ns + 1]
        cid = lax.axis_index("c")
        sid = lax.axis_index("s")
        wid = sid * NC + cid
        lane = lax.iota(jnp.int32, LANES)
        zero16 = jnp.zeros((LANES,), jnp.float32)

        def issue_idx(t, sl):
            idxs, idxd = sl[0], sl[1]
            si = sl[11]
            base = wid * e_per_w + t * b
            pltpu.async_copy(src_hbm.at[pl.ds(base, b)], idxs, si)
            pltpu.async_copy(dst_hbm.at[pl.ds(base, b)], idxd, si)

        def issue_gathers(t, sl):
            idxs, idxd, _, _, kg, qg, vg, peb = sl[:8]
            si, sk, sq, sv, sp = sl[11:16]
            base = wid * e_per_w + t * b
            pltpu.make_async_copy(src_hbm.at[pl.ds(0, b)], idxs, si).wait()
            pltpu.make_async_copy(dst_hbm.at[pl.ds(0, b)], idxd, si).wait()
            pltpu.async_copy(k_hbm.at[idxs], kg, sk)
            pltpu.async_copy(q_hbm.at[idxd], qg, sq)
            pltpu.async_copy(v_hbm.at[idxs], vg, sv)
            pltpu.async_copy(pe_hbm.at[pl.ds(base, b)], peb, sp)

        def wait_gathers(sl):
            idxs, idxd, _, _, kg, qg, vg, peb = sl[:8]
            sk, sq, sv, sp = sl[12:16]
            pltpu.make_async_copy(k_hbm.at[idxs], kg, sk).wait()
            pltpu.make_async_copy(q_hbm.at[idxd], qg, sq).wait()
            pltpu.make_async_copy(v_hbm.at[idxs], vg, sv).wait()
            pltpu.make_async_copy(pe_hbm.at[pl.ds(0, b)], peb, sp).wait()

        def issue_outputs(t, sl):
            dst_v, idxz, zrb, eoutb, msgb = sl[2], sl[3], sl[8], sl[9], sl[10]
            se, sm, sz = sl[16:19]
            base = wid * e_per_w + t * b
            pltpu.async_copy(eoutb, eout_hbm.at[pl.ds(base, b)], se)
            pltpu.async_copy(msgb, accum.at[dst_v], sm, add=True)
            pltpu.async_copy(zrb, zacc.at[idxz], sz, add=True)

        def wait_outputs(sl):
            dst_v, idxz, zrb, eoutb, msgb = sl[2], sl[3], sl[8], sl[9], sl[10]
            se, sm, sz = sl[16:19]
            pltpu.make_async_copy(eoutb, eout_hbm.at[pl.ds(0, b)], se).wait()
            pltpu.make_async_copy(msgb, accum.at[dst_v], sm).wait()
            pltpu.make_async_copy(zrb, zacc.at[idxz], sz).wait()

        def stage_idx(sl):
            _, idxd, dst_v, idxz = sl[:4]
            for g in range(b // LANES):
                g0 = g * LANES
                dd16 = idxd[pl.ds(g0, LANES)]
                dst_v[pl.ds(g0, LANES)] = dd16
                idxz[pl.ds(g0, LANES)] = jnp.right_shift(dd16, 4)

        def compute(sl):
            _, _, dst_v, idxz, kg, qg, vg, peb, zrb, eoutb, msgb = sl[:11]

            def group_body(g, _):
                g0 = g * LANES
                dd16 = dst_v[pl.ds(g0, LANES)]

                def edge_body(j, _):
                    i = g0 + j
                    zrow = zero16
                    for h in range(heads):
                        sl_ = pl.ds(d * h, d)
                        sc = jnp.clip(kg[i, sl_] * qg[i, sl_] * (1.0 / np.sqrt(d)),
                                      -5.0, 5.0) * peb[i, sl_]
                        tot = sc
                        for sh in (8, 4, 2, 1):
                            tot = tot + _lane_perm(tot, lane ^ sh)
                        eoutb[i, sl_] = sc
                        s = jnp.exp(jnp.clip(tot, -5.0, 5.0))
                        msgb[i, sl_] = vg[i, sl_] * s
                        zrow = jnp.where(lane == h, s, zrow)
                    dsel = _lane_perm_i32(dd16, jnp.full((LANES,), j, jnp.int32))
                    c = dsel & 15
                    cc = jnp.right_shift(c, 1)
                    odd = (c & 1) == 1
                    zhi = jnp.where(lane >= 8, _lane_perm(zrow, lane & 7), 0.0)
                    sel = jnp.where(odd, zhi, zrow)
                    for jc in range(hd // LANES):
                        zrb[i, pl.ds(LANES * jc, LANES)] = jnp.where(
                            cc == jc, sel, 0.0)
                    return 0

                lax.fori_loop(0, LANES, edge_body, 0)
                return 0

            lax.fori_loop(0, b // LANES, group_body, 0)

        peb0 = slots[0][7]

        def pb_zero(r, _):
            for j in range(hd // LANES):
                peb0[r, pl.ds(LANES * j, LANES)] = zero16
            return 0
        lax.fori_loop(0, b, pb_zero, 0)

        row0 = sid * rows_per_tile
        left = rows_per_tile
        while left > 0:
            step = min(b, left)
            pltpu.sync_copy(peb0.at[pl.ds(0, step)],
                            accum.at[pl.ds(row0 + rows_per_tile - left, step)])
            left -= step
        left = zrows_per_tile
        zoff = sid * zrows_per_tile
        while left > 0:
            step = min(b, left)
            pltpu.sync_copy(peb0.at[pl.ds(0, step)],
                            zacc.at[pl.ds(zoff + zrows_per_tile - left, step)])
            left -= step
        plsc.subcore_barrier()

        s0, s1 = slots
        issue_idx(0, s0)
        issue_idx(1, s1)
        issue_gathers(0, s0)

        wait_gathers(s0)
        stage_idx(s0)
        issue_idx(2, s0)
        issue_gathers(1, s1)
        compute(s0)
        issue_outputs(0, s0)
        wait_gathers(s1)
        stage_idx(s1)
        issue_idx(3, s1)
        wait_outputs(s0)
        issue_gathers(2, s0)
        compute(s1)
        issue_outputs(1, s1)

        def pair_body(p, _):
            t0 = 2 * p
            wait_outputs(s1)
            wait_gathers(s0)
            stage_idx(s0)
            issue_idx(t0 + 2, s0)
            issue_gathers(t0 + 1, s1)
            compute(s0)
            issue_outputs(t0, s0)
            wait_gathers(s1)
            stage_idx(s1)
            issue_idx(t0 + 3, s1)
            wait_outputs(s0)
            issue_gathers(t0 + 2, s0)
            compute(s1)
            issue_outputs(t0 + 1, s1)
            return 0

        lax.fori_loop(1, npairs - 1, pair_body, 0)

        wait_outputs(s1)
        wait_gathers(s0)
        stage_idx(s0)
        issue_gathers(nb - 1, s1)
        compute(s0)
        issue_outputs(nb - 2, s0)
        wait_gathers(s1)
        stage_idx(s1)
        wait_outputs(s0)
        compute(s1)
        issue_outputs(nb - 1, s1)
        wait_outputs(s1)
        plsc.subcore_barrier()

        pltpu.sync_copy(accum.at[pl.ds(row0, rows_per_tile)],
                        wv_hbm.at[cid, pl.ds(row0, rows_per_tile)])
        zr0 = sid * zrows_per_tile
        pltpu.sync_copy(zacc.at[pl.ds(zr0, zrows_per_tile)],
                        z_hbm.at[cid, pl.ds(zr0, zrows_per_tile)])

    return edge_kernel


def _epi_body(a_ref, b_ref, za_ref, zb_ref, ex_ref, o_ref):
    wv = a_ref[...] + b_ref[...]
    z = za_ref[...] + zb_ref[...]
    zrep = jnp.dot(z, ex_ref[...], preferred_element_type=jnp.float32)
    o_ref[...] = wv / (zrep + 1e-6)


def _epilogue(wv2, z2, n_pad, heads, d, bm):
    hd = heads * d
    expand = jnp.asarray(np.repeat(np.eye(heads, dtype=np.float32), d, axis=1))
    grid = n_pad // bm
    return pl.pallas_call(
        _epi_body,
        grid=(grid,),
        in_specs=[pl.BlockSpec((bm, hd), lambda i: (i, 0)),
                  pl.BlockSpec((bm, hd), lambda i, nb=grid: (i + nb, 0)),
                  pl.BlockSpec((bm, heads), lambda i: (i, 0)),
                  pl.BlockSpec((bm, heads), lambda i, nb=grid: (i + nb, 0)),
                  pl.BlockSpec((heads, hd), lambda i: (0, 0))],
        out_specs=pl.BlockSpec((bm, hd), lambda i: (i, 0)),
        out_shape=jax.ShapeDtypeStruct((n_pad, hd), jnp.float32),
    )(wv2, wv2, z2, z2, expand)


def kernel(node_feats, edge_feats, edge_index, Wq, Wk, Wv, We):
    n, f = node_feats.shape
    e = edge_feats.shape[0]
    hd = Wq.shape[0]
    heads, d = 8, hd // 8

    k_t, q_t, v_t = _node_proj(node_feats, Wk.T, Wq.T, Wv.T, bm=2000)
    pe = _edge_proj(edge_feats, We.T, bm=2000)

    b = 16
    nw = NC * NS
    e_per_w = -(-e // (nw * 2 * b)) * 2 * b
    e_pad = e_per_w * nw
    chunkn = NS * LANES * 8
    n_pad = ((n + chunkn - 1) // chunkn) * chunkn
    pad = e_pad - e
    src = jnp.concatenate([edge_index[0], jnp.zeros((pad,), edge_index.dtype)])
    dst = jnp.concatenate(
        [edge_index[1], jnp.full((pad,), n_pad - 1, edge_index.dtype)])
    pe = jnp.concatenate([pe, jnp.zeros((pad, hd), jnp.float32)])

    edge_kernel = _make_edge_kernel(n, e_pad, hd, heads, d, b=b)
    e_out, wv_part, z_part = edge_kernel(k_t, q_t, v_t, pe, src, dst)

    wv2 = wv_part.reshape(NC * n_pad, hd)
    z2 = z_part.reshape(NC * n_pad, heads)
    h_out = _epilogue(wv2, z2, n_pad, heads, d, bm=2048)

    return h_out[:n].reshape(n, heads, d), e_out[:e].reshape(e, heads, d)


# speedup vs baseline: 41.0640x; 1.1634x over previous
import functools

import jax
import jax.numpy as jnp
import numpy as np
from jax import lax
from jax.experimental import pallas as pl
from jax.experimental.pallas import tpu as pltpu
from jax.experimental.pallas import tpu_sc as plsc

NC = 2
NS = 16
LANES = 16

_GATHER_DNUMS = lax.GatherDimensionNumbers(
    offset_dims=(), collapsed_slice_dims=(0,), start_index_map=(0,))


def _lane_perm(v, idx):
    return lax.gather(v, idx[:, None], _GATHER_DNUMS, (1,),
                      mode=lax.GatherScatterMode.PROMISE_IN_BOUNDS)


def _lane_perm_i32(v, idx):
    return _lane_perm(v.view(jnp.float32), idx).view(jnp.int32)


def _proj3_body(x_ref, wk_ref, wq_ref, wv_ref, k_ref, q_ref, v_ref):
    x = x_ref[...]
    k_ref[...] = jnp.dot(x, wk_ref[...], preferred_element_type=jnp.float32)
    q_ref[...] = jnp.dot(x, wq_ref[...], preferred_element_type=jnp.float32)
    v_ref[...] = jnp.dot(x, wv_ref[...], preferred_element_type=jnp.float32)


def _proj1_body(x_ref, w_ref, o_ref):
    o_ref[...] = jnp.dot(x_ref[...], w_ref[...], preferred_element_type=jnp.float32)


def _node_proj(x, wkT, wqT, wvT, bm):
    r, f = x.shape
    hd = wkT.shape[1]
    grid = r // bm
    out = jax.ShapeDtypeStruct((r, hd), jnp.float32)
    wspec = pl.BlockSpec((f, hd), lambda i: (0, 0))
    return pl.pallas_call(
        _proj3_body,
        grid=(grid,),
        in_specs=[pl.BlockSpec((bm, f), lambda i: (i, 0)), wspec, wspec, wspec],
        out_specs=[pl.BlockSpec((bm, hd), lambda i: (i, 0))] * 3,
        out_shape=[out, out, out],
    )(x, wkT, wqT, wvT)


def _edge_proj(x, weT, bm):
    r, f = x.shape
    hd = weT.shape[1]
    grid = r // bm
    return pl.pallas_call(
        _proj1_body,
        grid=(grid,),
        in_specs=[pl.BlockSpec((bm, f), lambda i: (i, 0)),
                  pl.BlockSpec((f, hd), lambda i: (0, 0))],
        out_specs=pl.BlockSpec((bm, hd), lambda i: (i, 0)),
        out_shape=jax.ShapeDtypeStruct((r, hd), jnp.float32),
    )(x, weT)


def _make_edge_kernel(n, e, e_pad, hd, heads, d, b):
    e_per_w = e_pad // (NC * NS)
    real_per_w = e // (NC * NS)
    nb = real_per_w // b
    chunk = NS * LANES * 8
    n_pad = ((n + chunk - 1) // chunk) * chunk
    rows_per_tile = n_pad // NS
    zrows = n_pad // LANES
    zrows_per_tile = zrows // NS
    assert real_per_w % b == 0 and b % LANES == 0
    assert e_per_w == real_per_w + b and nb % 2 == 1 and nb >= 5
    assert zrows % NS == 0 and zrows_per_tile % 8 == 0

    npairs_main = (nb - 1) // 2

    mesh = plsc.VectorSubcoreMesh(core_axis_name="c", subcore_axis_name="s")

    slot_scratch = [
        pltpu.VMEM((b,), jnp.int32),
        pltpu.VMEM((b,), jnp.int32),
        pltpu.VMEM((b,), jnp.int32),
        pltpu.VMEM((b,), jnp.int32),
        pltpu.VMEM((b, hd), jnp.float32),
        pltpu.VMEM((b, hd), jnp.float32),
        pltpu.VMEM((b, hd), jnp.float32),
        pltpu.VMEM((b, hd), jnp.float32),
        pltpu.VMEM((b, hd), jnp.float32),
        pltpu.VMEM((b, hd), jnp.float32),
        pltpu.VMEM((b, hd), jnp.float32),
        pltpu.SemaphoreType.DMA,
        pltpu.SemaphoreType.DMA,
        pltpu.SemaphoreType.DMA,
        pltpu.SemaphoreType.DMA,
        pltpu.SemaphoreType.DMA,
        pltpu.SemaphoreType.DMA,
        pltpu.SemaphoreType.DMA,
        pltpu.SemaphoreType.DMA,
    ]

    @functools.partial(
        pl.kernel,
        out_type=(jax.ShapeDtypeStruct((e, hd), jnp.float32),
                  jax.ShapeDtypeStruct((NC, n_pad, hd), jnp.float32),
                  jax.ShapeDtypeStruct((NC, zrows, hd), jnp.float32)),
        mesh=mesh,
        compiler_params=pltpu.CompilerParams(needs_layout_passes=False),
        scratch_types=slot_scratch + slot_scratch + [
            pltpu.VMEM_SHARED((n_pad, hd), jnp.float32),
            pltpu.VMEM_SHARED((zrows, hd), jnp.float32),
        ],
    )
    def edge_kernel(k_hbm, q_hbm, v_hbm, pe_hbm, src_hbm, dst_hbm,
                    eout_hbm, wv_hbm, z_hbm, *scr):
        ns = 19
        slots = (scr[:ns], scr[ns:2 * ns])
        accum, zacc = scr[2 * ns], scr[2 * ns + 1]
        cid = lax.axis_index("c")
        sid = lax.axis_index("s")
        wid = sid * NC + cid
        lane = lax.iota(jnp.int32, LANES)
        zero16 = jnp.zeros((LANES,), jnp.float32)

        def issue_idx(t, sl):
            idxs, idxd = sl[0], sl[1]
            si = sl[11]
            base = wid * e_per_w + t * b
            pltpu.async_copy(src_hbm.at[pl.ds(base, b)], idxs, si)
            pltpu.async_copy(dst_hbm.at[pl.ds(base, b)], idxd, si)

        def drain_idx(sl):
            idxs, idxd = sl[0], sl[1]
            si = sl[11]
            pltpu.make_async_copy(src_hbm.at[pl.ds(0, b)], idxs, si).wait()
            pltpu.make_async_copy(dst_hbm.at[pl.ds(0, b)], idxd, si).wait()

        def issue_gathers(t, sl):
            idxs, idxd, _, _, kg, qg, vg, peb = sl[:8]
            sk, sq, sv, sp = sl[12:16]
            rbase = wid * real_per_w + t * b
            drain_idx(sl)
            pltpu.async_copy(k_hbm.at[idxs], kg, sk)
            pltpu.async_copy(q_hbm.at[idxd], qg, sq)
            pltpu.async_copy(v_hbm.at[idxs], vg, sv)
            pltpu.async_copy(pe_hbm.at[pl.ds(rbase, b)], peb, sp)

        def wait_gathers(sl):
            idxs, idxd, _, _, kg, qg, vg, peb = sl[:8]
            sk, sq, sv, sp = sl[12:16]
            pltpu.make_async_copy(k_hbm.at[idxs], kg, sk).wait()
            pltpu.make_async_copy(q_hbm.at[idxd], qg, sq).wait()
            pltpu.make_async_copy(v_hbm.at[idxs], vg, sv).wait()
            pltpu.make_async_copy(pe_hbm.at[pl.ds(0, b)], peb, sp).wait()

        def issue_outputs(t, sl):
            dst_v, idxz, zrb, eoutb, msgb = sl[2], sl[3], sl[8], sl[9], sl[10]
            se, sm, sz = sl[16:19]
            rbase = wid * real_per_w + t * b
            pltpu.async_copy(eoutb, eout_hbm.at[pl.ds(rbase, b)], se)
            pltpu.async_copy(msgb, accum.at[dst_v], sm, add=True)
            pltpu.async_copy(zrb, zacc.at[idxz], sz, add=True)

        def wait_outputs(sl):
            dst_v, idxz, zrb, eoutb, msgb = sl[2], sl[3], sl[8], sl[9], sl[10]
            se, sm, sz = sl[16:19]
            pltpu.make_async_copy(eoutb, eout_hbm.at[pl.ds(0, b)], se).wait()
            pltpu.make_async_copy(msgb, accum.at[dst_v], sm).wait()
            pltpu.make_async_copy(zrb, zacc.at[idxz], sz).wait()

        def stage_idx(sl):
            _, idxd, dst_v, idxz = sl[:4]
            for g in range(b // LANES):
                g0 = g * LANES
                dd16 = idxd[pl.ds(g0, LANES)]
                dst_v[pl.ds(g0, LANES)] = dd16
                idxz[pl.ds(g0, LANES)] = jnp.right_shift(dd16, 4)

        def compute(sl):
            _, _, dst_v, idxz, kg, qg, vg, peb, zrb, eoutb, msgb = sl[:11]

            def group_body(g, _):
                g0 = g * LANES
                dd16 = dst_v[pl.ds(g0, LANES)]

                def edge_body(j, _):
                    i = g0 + j
                    zrow = zero16
                    for h in range(heads):
                        sl_ = pl.ds(d * h, d)
                        sc = jnp.clip(kg[i, sl_] * qg[i, sl_] * (1.0 / np.sqrt(d)),
                                      -5.0, 5.0) * peb[i, sl_]
                        tot = sc
                        for sh in (8, 4, 2, 1):
                            tot = tot + _lane_perm(tot, lane ^ sh)
                        eoutb[i, sl_] = sc
                        s = jnp.exp(jnp.clip(tot, -5.0, 5.0))
                        msgb[i, sl_] = vg[i, sl_] * s
                        zrow = jnp.where(lane == h, s, zrow)
                    dsel = _lane_perm_i32(dd16, jnp.full((LANES,), j, jnp.int32))
                    c = dsel & 15
                    cc = jnp.right_shift(c, 1)
                    odd = (c & 1) == 1
                    zhi = jnp.where(lane >= 8, _lane_perm(zrow, lane & 7), 0.0)
                    sel = jnp.where(odd, zhi, zrow)
                    for jc in range(hd // LANES):
                        zrb[i, pl.ds(LANES * jc, LANES)] = jnp.where(
                            cc == jc, sel, 0.0)
                    return 0

                lax.fori_loop(0, LANES, edge_body, 0)
                return 0

            lax.fori_loop(0, b // LANES, group_body, 0)

        peb0 = slots[0][7]

        def pb_zero(r, _):
            for j in range(hd // LANES):
                peb0[r, pl.ds(LANES * j, LANES)] = zero16
            return 0
        lax.fori_loop(0, b, pb_zero, 0)

        row0 = sid * rows_per_tile
        left = rows_per_tile
        while left > 0:
            step = min(b, left)
            pltpu.sync_copy(peb0.at[pl.ds(0, step)],
                            accum.at[pl.ds(row0 + rows_per_tile - left, step)])
            left -= step
        left = zrows_per_tile
        zoff = sid * zrows_per_tile
        while left > 0:
            step = min(b, left)
            pltpu.sync_copy(peb0.at[pl.ds(0, step)],
                            zacc.at[pl.ds(zoff + zrows_per_tile - left, step)])
            left -= step
        plsc.subcore_barrier()

        s0, s1 = slots
        issue_idx(0, s0)
        issue_idx(1, s1)
        issue_gathers(0, s0)

        wait_gathers(s0)
        stage_idx(s0)
        issue_idx(2, s0)
        issue_gathers(1, s1)
        compute(s0)
        issue_outputs(0, s0)
        wait_gathers(s1)
        stage_idx(s1)
        issue_idx(3, s1)
        wait_outputs(s0)
        issue_gathers(2, s0)
        compute(s1)
        issue_outputs(1, s1)

        def pair_body(p, _):
            t0 = 2 * p
            wait_outputs(s1)
            wait_gathers(s0)
            stage_idx(s0)
            issue_idx(t0 + 2, s0)
            issue_gathers(t0 + 1, s1)
            compute(s0)
            issue_outputs(t0, s0)
            wait_gathers(s1)
            stage_idx(s1)
            issue_idx(t0 + 3, s1)
            wait_outputs(s0)
            issue_gathers(t0 + 2, s0)
            compute(s1)
            issue_outputs(t0 + 1, s1)
            return 0

        lax.fori_loop(1, npairs_main, pair_body, 0)

        wait_outputs(s1)
        wait_gathers(s0)
        stage_idx(s0)
        compute(s0)
        issue_outputs(nb - 1, s0)
        drain_idx(s1)
        wait_outputs(s0)
        plsc.subcore_barrier()

        pltpu.sync_copy(accum.at[pl.ds(row0, rows_per_tile)],
                        wv_hbm.at[cid, pl.ds(row0, rows_per_tile)])
        zr0 = sid * zrows_per_tile
        pltpu.sync_copy(zacc.at[pl.ds(zr0, zrows_per_tile)],
                        z_hbm.at[cid, pl.ds(zr0, zrows_per_tile)])

    return edge_kernel


def _epi_body(a_ref, b_ref, za_ref, zb_ref, ex_ref, o_ref):
    wv = a_ref[...] + b_ref[...]
    z = za_ref[...] + zb_ref[...]
    zrep = jnp.dot(z, ex_ref[...], preferred_element_type=jnp.float32)
    o_ref[...] = wv / (zrep + 1e-6)


def _epilogue(wv2, z2, n_pad, heads, d, bm):
    hd = heads * d
    expand = jnp.asarray(np.repeat(np.eye(heads, dtype=np.float32), d, axis=1))
    grid = n_pad // bm
    return pl.pallas_call(
        _epi_body,
        grid=(grid,),
        in_specs=[pl.BlockSpec((bm, hd), lambda i: (i, 0)),
                  pl.BlockSpec((bm, hd), lambda i, nb=grid: (i + nb, 0)),
                  pl.BlockSpec((bm, heads), lambda i: (i, 0)),
                  pl.BlockSpec((bm, heads), lambda i, nb=grid: (i + nb, 0)),
                  pl.BlockSpec((heads, hd), lambda i: (0, 0))],
        out_specs=pl.BlockSpec((bm, hd), lambda i: (i, 0)),
        out_shape=jax.ShapeDtypeStruct((n_pad, hd), jnp.float32),
    )(wv2, wv2, z2, z2, expand)


def kernel(node_feats, edge_feats, edge_index, Wq, Wk, Wv, We):
    n, f = node_feats.shape
    e = edge_feats.shape[0]
    hd = Wq.shape[0]
    heads, d = 8, hd // 8

    k_t, q_t, v_t = _node_proj(node_feats, Wk.T, Wq.T, Wv.T, bm=2000)
    pe = _edge_proj(edge_feats, We.T, bm=2000)

    b = 16
    nw = NC * NS
    assert e % nw == 0 and (e // nw) % b == 0
    real_per_w = e // nw
    chunkn = NS * LANES * 8
    n_pad = ((n + chunkn - 1) // chunkn) * chunkn
    src = jnp.pad(edge_index[0].reshape(nw, real_per_w),
                  ((0, 0), (0, b))).reshape(-1)
    dst = jnp.pad(edge_index[1].reshape(nw, real_per_w), ((0, 0), (0, b)),
                  constant_values=n_pad - 1).reshape(-1)
    e_pad = e + nw * b

    edge_kernel = _make_edge_kernel(n, e, e_pad, hd, heads, d, b=b)
    e_out, wv_part, z_part = edge_kernel(k_t, q_t, v_t, pe, src, dst)

    wv2 = wv_part.reshape(NC * n_pad, hd)
    z2 = z_part.reshape(NC * n_pad, heads)
    h_out = _epilogue(wv2, z2, n_pad, heads, d, bm=2048)

    return h_out[:n].reshape(n, heads, d), e_out.reshape(e, heads, d)


# speedup vs baseline: 41.1375x; 1.0018x over previous
import functools

import jax
import jax.numpy as jnp
import numpy as np
from jax import lax
from jax.experimental import pallas as pl
from jax.experimental.pallas import tpu as pltpu
from jax.experimental.pallas import tpu_sc as plsc

NC = 2
NS = 16
LANES = 16

_GATHER_DNUMS = lax.GatherDimensionNumbers(
    offset_dims=(), collapsed_slice_dims=(0,), start_index_map=(0,))


def _lane_perm(v, idx):
    return lax.gather(v, idx[:, None], _GATHER_DNUMS, (1,),
                      mode=lax.GatherScatterMode.PROMISE_IN_BOUNDS)


def _lane_perm_i32(v, idx):
    return _lane_perm(v.view(jnp.float32), idx).view(jnp.int32)


def _make_proj_body(ngrid):
    def _proj_body(ef_ref, nf_ref, wk_ref, wq_ref, wv_ref, we_ref,
                   pe_ref, k_ref, q_ref, v_ref):
        pe_ref[...] = jnp.dot(ef_ref[...], we_ref[...],
                              preferred_element_type=jnp.float32)

        @pl.when(pl.program_id(0) < ngrid)
        def _():
            x = nf_ref[...]
            k_ref[...] = jnp.dot(x, wk_ref[...],
                                 preferred_element_type=jnp.float32)
            q_ref[...] = jnp.dot(x, wq_ref[...],
                                 preferred_element_type=jnp.float32)
            v_ref[...] = jnp.dot(x, wv_ref[...],
                                 preferred_element_type=jnp.float32)

    return _proj_body


def _projections(ef, nf, wkT, wqT, wvT, weT, bm):
    e, f = ef.shape
    r = nf.shape[0]
    hd = weT.shape[1]
    grid = e // bm
    ngrid = r // bm
    assert e % bm == 0 and r % bm == 0 and ngrid <= grid
    nout = jax.ShapeDtypeStruct((r, hd), jnp.float32)
    wspec = pl.BlockSpec((f, hd), lambda i: (0, 0))

    def nmap(i):
        return (jnp.minimum(i, ngrid - 1), 0)

    return pl.pallas_call(
        _make_proj_body(ngrid),
        grid=(grid,),
        in_specs=[pl.BlockSpec((bm, f), lambda i: (i, 0)),
                  pl.BlockSpec((bm, f), nmap),
                  wspec, wspec, wspec, wspec],
        out_specs=[pl.BlockSpec((bm, hd), lambda i: (i, 0))] +
                  [pl.BlockSpec((bm, hd), nmap)] * 3,
        out_shape=[jax.ShapeDtypeStruct((e, hd), jnp.float32),
                   nout, nout, nout],
    )(ef, nf, wkT, wqT, wvT, weT)


def _make_edge_kernel(n, e, e_pad, hd, heads, d, b):
    e_per_w = e_pad // (NC * NS)
    real_per_w = e // (NC * NS)
    nb = real_per_w // b
    chunk = NS * LANES * 8
    n_pad = ((n + chunk - 1) // chunk) * chunk
    rows_per_tile = n_pad // NS
    zrows = n_pad // LANES
    zrows_per_tile = zrows // NS
    assert real_per_w % b == 0 and b % LANES == 0
    assert e_per_w == real_per_w + b and nb % 2 == 1 and nb >= 5
    assert zrows % NS == 0 and zrows_per_tile % 8 == 0

    npairs_main = (nb - 1) // 2

    mesh = plsc.VectorSubcoreMesh(core_axis_name="c", subcore_axis_name="s")

    slot_scratch = [
        pltpu.VMEM((b,), jnp.int32),
        pltpu.VMEM((b,), jnp.int32),
        pltpu.VMEM((b,), jnp.int32),
        pltpu.VMEM((b,), jnp.int32),
        pltpu.VMEM((b, hd), jnp.float32),
        pltpu.VMEM((b, hd), jnp.float32),
        pltpu.VMEM((b, hd), jnp.float32),
        pltpu.VMEM((b, hd), jnp.float32),
        pltpu.VMEM((b, hd), jnp.float32),
        pltpu.VMEM((b, hd), jnp.float32),
        pltpu.VMEM((b, hd), jnp.float32),
        pltpu.SemaphoreType.DMA,
        pltpu.SemaphoreType.DMA,
        pltpu.SemaphoreType.DMA,
        pltpu.SemaphoreType.DMA,
        pltpu.SemaphoreType.DMA,
        pltpu.SemaphoreType.DMA,
        pltpu.SemaphoreType.DMA,
        pltpu.SemaphoreType.DMA,
    ]

    @functools.partial(
        pl.kernel,
        out_type=(jax.ShapeDtypeStruct((e, hd), jnp.float32),
                  jax.ShapeDtypeStruct((NC, n_pad, hd), jnp.float32),
                  jax.ShapeDtypeStruct((NC, zrows, hd), jnp.float32)),
        mesh=mesh,
        compiler_params=pltpu.CompilerParams(needs_layout_passes=False),
        scratch_types=slot_scratch + slot_scratch + [
            pltpu.VMEM_SHARED((n_pad, hd), jnp.float32),
            pltpu.VMEM_SHARED((zrows, hd), jnp.float32),
        ],
    )
    def edge_kernel(k_hbm, q_hbm, v_hbm, pe_hbm, src_hbm, dst_hbm,
                    eout_hbm, wv_hbm, z_hbm, *scr):
        ns = 19
        slots = (scr[:ns], scr[ns:2 * ns])
        accum, zacc = scr[2 * ns], scr[2 * ns + 1]
        cid = lax.axis_index("c")
        sid = lax.axis_index("s")
        wid = sid * NC + cid
        lane = lax.iota(jnp.int32, LANES)
        zero16 = jnp.zeros((LANES,), jnp.float32)

        def issue_idx(t, sl):
            idxs, idxd = sl[0], sl[1]
            si = sl[11]
            base = wid * e_per_w + t * b
            pltpu.async_copy(src_hbm.at[pl.ds(base, b)], idxs, si)
            pltpu.async_copy(dst_hbm.at[pl.ds(base, b)], idxd, si)

        def drain_idx(sl):
            idxs, idxd = sl[0], sl[1]
            si = sl[11]
            pltpu.make_async_copy(src_hbm.at[pl.ds(0, b)], idxs, si).wait()
            pltpu.make_async_copy(dst_hbm.at[pl.ds(0, b)], idxd, si).wait()

        def issue_gathers(t, sl):
            idxs, idxd, _, _, kg, qg, vg, peb = sl[:8]
            sk, sq, sv, sp = sl[12:16]
            rbase = wid * real_per_w + t * b
            drain_idx(sl)
            pltpu.async_copy(k_hbm.at[idxs], kg, sk)
            pltpu.async_copy(q_hbm.at[idxd], qg, sq)
            pltpu.async_copy(v_hbm.at[idxs], vg, sv)
            pltpu.async_copy(pe_hbm.at[pl.ds(rbase, b)], peb, sp)

        def wait_gathers(sl):
            idxs, idxd, _, _, kg, qg, vg, peb = sl[:8]
            sk, sq, sv, sp = sl[12:16]
            pltpu.make_async_copy(k_hbm.at[idxs], kg, sk).wait()
            pltpu.make_async_copy(q_hbm.at[idxd], qg, sq).wait()
            pltpu.make_async_copy(v_hbm.at[idxs], vg, sv).wait()
            pltpu.make_async_copy(pe_hbm.at[pl.ds(0, b)], peb, sp).wait()

        def issue_outputs(t, sl):
            dst_v, idxz, zrb, eoutb, msgb = sl[2], sl[3], sl[8], sl[9], sl[10]
            se, sm, sz = sl[16:19]
            rbase = wid * real_per_w + t * b
            pltpu.async_copy(eoutb, eout_hbm.at[pl.ds(rbase, b)], se)
            pltpu.async_copy(msgb, accum.at[dst_v], sm, add=True)
            pltpu.async_copy(zrb, zacc.at[idxz], sz, add=True)

        def wait_outputs(sl):
            dst_v, idxz, zrb, eoutb, msgb = sl[2], sl[3], sl[8], sl[9], sl[10]
            se, sm, sz = sl[16:19]
            pltpu.make_async_copy(eoutb, eout_hbm.at[pl.ds(0, b)], se).wait()
            pltpu.make_async_copy(msgb, accum.at[dst_v], sm).wait()
            pltpu.make_async_copy(zrb, zacc.at[idxz], sz).wait()

        def stage_idx(sl):
            _, idxd, dst_v, idxz = sl[:4]
            for g in range(b // LANES):
                g0 = g * LANES
                dd16 = idxd[pl.ds(g0, LANES)]
                dst_v[pl.ds(g0, LANES)] = dd16
                idxz[pl.ds(g0, LANES)] = jnp.right_shift(dd16, 4)

        def compute(sl):
            _, _, dst_v, idxz, kg, qg, vg, peb, zrb, eoutb, msgb = sl[:11]

            def group_body(g, _):
                g0 = g * LANES
                dd16 = dst_v[pl.ds(g0, LANES)]

                def edge_body(j, _):
                    i = g0 + j
                    zrow = zero16
                    for h in range(heads):
                        sl_ = pl.ds(d * h, d)
                        sc = jnp.clip(kg[i, sl_] * qg[i, sl_] * (1.0 / np.sqrt(d)),
                                      -5.0, 5.0) * peb[i, sl_]
                        tot = sc
                        for sh in (8, 4, 2, 1):
                            tot = tot + _lane_perm(tot, lane ^ sh)
                        eoutb[i, sl_] = sc
                        s = jnp.exp(jnp.clip(tot, -5.0, 5.0))
                        msgb[i, sl_] = vg[i, sl_] * s
                        zrow = jnp.where(lane == h, s, zrow)
                    dsel = _lane_perm_i32(dd16, jnp.full((LANES,), j, jnp.int32))
                    c = dsel & 15
                    cc = jnp.right_shift(c, 1)
                    odd = (c & 1) == 1
                    zhi = jnp.where(lane >= 8, _lane_perm(zrow, lane & 7), 0.0)
                    sel = jnp.where(odd, zhi, zrow)
                    for jc in range(hd // LANES):
                        zrb[i, pl.ds(LANES * jc, LANES)] = jnp.where(
                            cc == jc, sel, 0.0)
                    return 0

                lax.fori_loop(0, LANES, edge_body, 0)
                return 0

            lax.fori_loop(0, b // LANES, group_body, 0)

        peb0 = slots[0][7]

        def pb_zero(r, _):
            for j in range(hd // LANES):
                peb0[r, pl.ds(LANES * j, LANES)] = zero16
            return 0
        lax.fori_loop(0, b, pb_zero, 0)

        row0 = sid * rows_per_tile
        left = rows_per_tile
        while left > 0:
            step = min(b, left)
            pltpu.sync_copy(peb0.at[pl.ds(0, step)],
                            accum.at[pl.ds(row0 + rows_per_tile - left, step)])
            left -= step
        left = zrows_per_tile
        zoff = sid * zrows_per_tile
        while left > 0:
            step = min(b, left)
            pltpu.sync_copy(peb0.at[pl.ds(0, step)],
                            zacc.at[pl.ds(zoff + zrows_per_tile - left, step)])
            left -= step
        plsc.subcore_barrier()

        s0, s1 = slots
        issue_idx(0, s0)
        issue_idx(1, s1)
        issue_gathers(0, s0)

        wait_gathers(s0)
        stage_idx(s0)
        issue_idx(2, s0)
        issue_gathers(1, s1)
        compute(s0)
        issue_outputs(0, s0)
        wait_gathers(s1)
        stage_idx(s1)
        issue_idx(3, s1)
        wait_outputs(s0)
        issue_gathers(2, s0)
        compute(s1)
        issue_outputs(1, s1)

        def pair_body(p, _):
            t0 = 2 * p
            wait_outputs(s1)
            wait_gathers(s0)
            stage_idx(s0)
            issue_idx(t0 + 2, s0)
            issue_gathers(t0 + 1, s1)
            compute(s0)
            issue_outputs(t0, s0)
            wait_gathers(s1)
            stage_idx(s1)
            issue_idx(t0 + 3, s1)
            wait_outputs(s0)
            issue_gathers(t0 + 2, s0)
            compute(s1)
            issue_outputs(t0 + 1, s1)
            return 0

        lax.fori_loop(1, npairs_main, pair_body, 0)

        wait_outputs(s1)
        wait_gathers(s0)
        stage_idx(s0)
        compute(s0)
        issue_outputs(nb - 1, s0)
        drain_idx(s1)
        wait_outputs(s0)
        plsc.subcore_barrier()

        pltpu.sync_copy(accum.at[pl.ds(row0, rows_per_tile)],
                        wv_hbm.at[cid, pl.ds(row0, rows_per_tile)])
        zr0 = sid * zrows_per_tile
        pltpu.sync_copy(zacc.at[pl.ds(zr0, zrows_per_tile)],
                        z_hbm.at[cid, pl.ds(zr0, zrows_per_tile)])

    return edge_kernel


def _epi_body(a_ref, b_ref, za_ref, zb_ref, ex_ref, o_ref):
    wv = a_ref[...] + b_ref[...]
    z = za_ref[...] + zb_ref[...]
    zrep = jnp.dot(z, ex_ref[...], preferred_element_type=jnp.float32)
    o_ref[...] = wv / (zrep + 1e-6)


def _epilogue(wv2, z2, n_pad, heads, d, bm):
    hd = heads * d
    expand = jnp.asarray(np.repeat(np.eye(heads, dtype=np.float32), d, axis=1))
    grid = n_pad // bm
    return pl.pallas_call(
        _epi_body,
        grid=(grid,),
        in_specs=[pl.BlockSpec((bm, hd), lambda i: (i, 0)),
                  pl.BlockSpec((bm, hd), lambda i, nb=grid: (i + nb, 0)),
                  pl.BlockSpec((bm, heads), lambda i: (i, 0)),
                  pl.BlockSpec((bm, heads), lambda i, nb=grid: (i + nb, 0)),
                  pl.BlockSpec((heads, hd), lambda i: (0, 0))],
        out_specs=pl.BlockSpec((bm, hd), lambda i: (i, 0)),
        out_shape=jax.ShapeDtypeStruct((n_pad, hd), jnp.float32),
    )(wv2, wv2, z2, z2, expand)


def kernel(node_feats, edge_feats, edge_index, Wq, Wk, Wv, We):
    n, f = node_feats.shape
    e = edge_feats.shape[0]
    hd = Wq.shape[0]
    heads, d = 8, hd // 8

    pe, k_t, q_t, v_t = _projections(
        edge_feats, node_feats, Wk.T, Wq.T, Wv.T, We.T, bm=2000)

    b = 16
    nw = NC * NS
    assert e % nw == 0 and (e // nw) % b == 0
    real_per_w = e // nw
    chunkn = NS * LANES * 8
    n_pad = ((n + chunkn - 1) // chunkn) * chunkn
    src = jnp.pad(edge_index[0].reshape(nw, real_per_w),
                  ((0, 0), (0, b))).reshape(-1)
    dst = jnp.pad(edge_index[1].reshape(nw, real_per_w), ((0, 0), (0, b)),
                  constant_values=n_pad - 1).reshape(-1)
    e_pad = e + nw * b

    edge_kernel = _make_edge_kernel(n, e, e_pad, hd, heads, d, b=b)
    e_out, wv_part, z_part = edge_kernel(k_t, q_t, v_t, pe, src, dst)

    wv2 = wv_part.reshape(NC * n_pad, hd)
    z2 = z_part.reshape(NC * n_pad, heads)
    h_out = _epilogue(wv2, z2, n_pad, heads, d, bm=2048)

    return h_out[:n].reshape(n, heads, d), e_out.reshape(e, heads, d)
